# baseline (speedup 1.0000x reference)
import functools

import jax
import jax.numpy as jnp
from jax import lax
from jax.experimental import pallas as pl
from jax.experimental.pallas import tpu as pltpu
from jax.experimental.pallas import tpu_sc as plsc

EMB_D = 64
NUM_E = 1024
VQ_BETA = 0.25
ROWS = 32 * 576
TILE = 3072
NSTEP = ROWS // TILE
LOSS_SCALE = VQ_BETA / float(ROWS * EMB_D)
PAD_D = 128

NW = 32
BPW = ROWS // NW
CH = BPW // 2


def _tc_body(x_ref, emb_ref, idx_ref, loss_ref, pad_ref):
    i = pl.program_id(0)
    x = x_ref[...]
    emb = emb_ref[...]
    xsq = jnp.sum(x * x, axis=1, keepdims=True)
    esq = jnp.sum(emb * emb, axis=1)[None, :]
    m = lax.dot_general(x, emb, (((1,), (1,)), ((), ())),
                        preferred_element_type=jnp.float32)
    d = (xsq + esq) - 2.0 * m
    mind = jnp.min(d, axis=1, keepdims=True)
    nchunk = NUM_E // 128
    firstc = None
    for c in range(nchunk):
        cc = jnp.where(d[:, c * 128:(c + 1) * 128] == mind, c, nchunk)
        firstc = cc if firstc is None else jnp.minimum(firstc, cc)
    lane = lax.broadcasted_iota(jnp.int32, (TILE, 128), 1)
    key = jnp.where(firstc < nchunk, firstc * 128 + lane, NUM_E)
    idx_ref[...] = jnp.min(key, axis=1, keepdims=True)

    pad_ref[:, :EMB_D] = emb
    pad_ref[:, EMB_D:] = jnp.zeros((NUM_E, PAD_D - EMB_D), jnp.float32)

    part = jnp.sum(mind)

    @pl.when(i == 0)
    def _():
        loss_ref[0, 0] = 0.0

    loss_ref[0, 0] += part

    @pl.when(i == NSTEP - 1)
    def _():
        loss_ref[0, 0] *= LOSS_SCALE


@functools.cache
def _make_sc_gather():
    mesh = plsc.VectorSubcoreMesh(core_axis_name="c", subcore_axis_name="s")

    @functools.partial(
        pl.kernel,
        mesh=mesh,
        out_type=jax.ShapeDtypeStruct((ROWS, PAD_D), jnp.float32),
        scratch_types=[
            pltpu.VMEM((CH,), jnp.int32),
            pltpu.VMEM((CH,), jnp.int32),
            pltpu.VMEM((CH, PAD_D), jnp.float32),
            pltpu.VMEM((CH, PAD_D), jnp.float32),
            pltpu.SemaphoreType.DMA,
            pltpu.SemaphoreType.DMA,
        ],
    )
    def _sc_gather(table_hbm, idx_hbm, out_hbm,
                   idx_a, idx_b, rows_a, rows_b, sem_a, sem_b):
        wid = lax.axis_index("s") * 2 + lax.axis_index("c")
        base = wid * BPW
        pltpu.sync_copy(idx_hbm.at[pl.ds(base, CH)], idx_a)
        cp_a = pltpu.async_copy(table_hbm.at[idx_a], rows_a, sem_a)
        pltpu.sync_copy(idx_hbm.at[pl.ds(base + CH, CH)], idx_b)
        cp_b = pltpu.async_copy(table_hbm.at[idx_b], rows_b, sem_b)
        cp_a.wait()
        pltpu.sync_copy(rows_a, out_hbm.at[pl.ds(base, CH)])
        cp_b.wait()
        pltpu.sync_copy(rows_b, out_hbm.at[pl.ds(base + CH, CH)])

    return _sc_gather


def kernel(x, embeddings):
    flat_x = x.reshape(-1, EMB_D)
    idx2d, loss_sum, table_pad = pl.pallas_call(
        _tc_body,
        grid=(NSTEP,),
        in_specs=[
            pl.BlockSpec((TILE, EMB_D), lambda i: (i, 0)),
            pl.BlockSpec((NUM_E, EMB_D), lambda i: (0, 0)),
        ],
        out_specs=[
            pl.BlockSpec((TILE, 1), lambda i: (i, 0)),
            pl.BlockSpec((1, 1), lambda i: (0, 0), memory_space=pltpu.SMEM),
            pl.BlockSpec((NUM_E, PAD_D), lambda i: (0, 0)),
        ],
        out_shape=[
            jax.ShapeDtypeStruct((ROWS, 1), jnp.int32),
            jax.ShapeDtypeStruct((1, 1), jnp.float32),
            jax.ShapeDtypeStruct((NUM_E, PAD_D), jnp.float32),
        ],
    )(flat_x, embeddings)
    idx = idx2d.reshape(ROWS)
    q = _make_sc_gather()(table_pad, idx)[:, :EMB_D]
    return idx, q.reshape(x.shape), loss_sum[0, 0]
